# baseline (speedup 1.0000x reference)
import jax
import jax.numpy as jnp
from jax.experimental import pallas as pl
from jax.experimental.pallas import tpu as pltpu


def _fused_heads_body(x_ref, w_ref, b_ref, out_ref):
    out_ref[...] = (
        jnp.dot(x_ref[...], w_ref[...], preferred_element_type=jnp.float32)
        + b_ref[...]
    )


def kernel(x, W_cls, b_cls, W_box, b_box):
    if x.ndim > 2:
        x = x.reshape(x.shape[0], -1)
    N, D = x.shape
    C = W_cls.shape[0]
    B = W_box.shape[0]

    W = jnp.concatenate([W_cls, W_box], axis=0)
    Wp = jnp.zeros((128, D), x.dtype).at[: C + B].set(W).T
    bp = (
        jnp.zeros((1, 128), x.dtype)
        .at[0, :C].set(b_cls)
        .at[0, C : C + B].set(b_box)
    )

    NB = 1000
    pad = (-N) % NB
    if pad:
        x = jnp.pad(x, ((0, pad), (0, 0)))
    Np = N + pad

    out = pl.pallas_call(
        _fused_heads_body,
        grid=(Np // NB,),
        in_specs=[
            pl.BlockSpec((NB, D), lambda i: (i, 0)),
            pl.BlockSpec((D, 128), lambda i: (0, 0)),
            pl.BlockSpec((1, 128), lambda i: (0, 0)),
        ],
        out_specs=pl.BlockSpec((NB, 128), lambda i: (i, 0)),
        out_shape=jax.ShapeDtypeStruct((Np, 128), jnp.float32),
        compiler_params=pltpu.CompilerParams(
            dimension_semantics=("arbitrary",),
        ),
    )(x, Wp, bp)

    return out[:N, :C], out[:N, C : C + B]


# speedup vs baseline: 1.2005x; 1.2005x over previous
import jax
import jax.numpy as jnp
from jax.experimental import pallas as pl
from jax.experimental.pallas import tpu as pltpu


def _fused_heads_body(x_ref, w_ref, b_ref, scores_ref, deltas_ref):
    C = scores_ref.shape[-1]
    B = deltas_ref.shape[-1]
    r = (
        jnp.dot(x_ref[...], w_ref[...], preferred_element_type=jnp.float32)
        + b_ref[...]
    )
    scores_ref[...] = r[:, :C]
    deltas_ref[...] = r[:, C : C + B]


def kernel(x, W_cls, b_cls, W_box, b_box):
    if x.ndim > 2:
        x = x.reshape(x.shape[0], -1)
    N, D = x.shape
    C = W_cls.shape[0]
    B = W_box.shape[0]

    W = jnp.concatenate([W_cls, W_box], axis=0)
    Wp = jnp.zeros((128, D), x.dtype).at[: C + B].set(W).T
    bp = (
        jnp.zeros((1, 128), x.dtype)
        .at[0, :C].set(b_cls)
        .at[0, C : C + B].set(b_box)
    )

    NB = 2000
    pad = (-N) % NB
    if pad:
        x = jnp.pad(x, ((0, pad), (0, 0)))
    Np = N + pad

    scores, deltas = pl.pallas_call(
        _fused_heads_body,
        grid=(Np // NB,),
        in_specs=[
            pl.BlockSpec((NB, D), lambda i: (i, 0)),
            pl.BlockSpec((D, 128), lambda i: (0, 0)),
            pl.BlockSpec((1, 128), lambda i: (0, 0)),
        ],
        out_specs=[
            pl.BlockSpec((NB, C), lambda i: (i, 0)),
            pl.BlockSpec((NB, B), lambda i: (i, 0)),
        ],
        out_shape=[
            jax.ShapeDtypeStruct((Np, C), jnp.float32),
            jax.ShapeDtypeStruct((Np, B), jnp.float32),
        ],
        compiler_params=pltpu.CompilerParams(
            dimension_semantics=("arbitrary",),
        ),
    )(x, Wp, bp)

    if pad:
        scores, deltas = scores[:N], deltas[:N]
    return scores, deltas


# speedup vs baseline: 1.2044x; 1.0033x over previous
import jax
import jax.numpy as jnp
from jax.experimental import pallas as pl
from jax.experimental.pallas import tpu as pltpu


def _fused_heads_body(x_ref, w_ref, b_ref, scores_ref, deltas_ref):
    C = scores_ref.shape[-1]
    B = deltas_ref.shape[-1]
    r = (
        jnp.dot(x_ref[...], w_ref[...], preferred_element_type=jnp.float32)
        + b_ref[...]
    )
    scores_ref[...] = r[:, :C]
    deltas_ref[...] = r[:, C : C + B]


def kernel(x, W_cls, b_cls, W_box, b_box):
    if x.ndim > 2:
        x = x.reshape(x.shape[0], -1)
    N, D = x.shape
    C = W_cls.shape[0]
    B = W_box.shape[0]

    W = jnp.concatenate([W_cls, W_box], axis=0)
    Wp = jnp.zeros((128, D), x.dtype).at[: C + B].set(W).T
    bp = (
        jnp.zeros((1, 128), x.dtype)
        .at[0, :C].set(b_cls)
        .at[0, C : C + B].set(b_box)
    )

    NB = 4000
    pad = (-N) % NB
    if pad:
        x = jnp.pad(x, ((0, pad), (0, 0)))
    Np = N + pad

    scores, deltas = pl.pallas_call(
        _fused_heads_body,
        grid=(Np // NB,),
        in_specs=[
            pl.BlockSpec((NB, D), lambda i: (i, 0)),
            pl.BlockSpec((D, 128), lambda i: (0, 0)),
            pl.BlockSpec((1, 128), lambda i: (0, 0)),
        ],
        out_specs=[
            pl.BlockSpec((NB, C), lambda i: (i, 0)),
            pl.BlockSpec((NB, B), lambda i: (i, 0)),
        ],
        out_shape=[
            jax.ShapeDtypeStruct((Np, C), jnp.float32),
            jax.ShapeDtypeStruct((Np, B), jnp.float32),
        ],
        compiler_params=pltpu.CompilerParams(
            dimension_semantics=("arbitrary",),
        ),
    )(x, Wp, bp)

    if pad:
        scores, deltas = scores[:N], deltas[:N]
    return scores, deltas
